# baseline (speedup 1.0000x reference)
import functools

import jax
import jax.numpy as jnp
from jax import lax
from jax.experimental import pallas as pl
from jax.experimental.pallas import tpu as pltpu
from jax.experimental.pallas import tpu_sc as plsc

VOCAB = 1_000_000
EMB = 32
B = 16384
T = 50
CHUNK = 1024
N_CHUNKS = B // CHUNK
N_UNITS = T * N_CHUNKS
RW = 128
NSTREAM = CHUNK // RW


@functools.lru_cache(maxsize=1)
def _build():
    info = plsc.get_sparse_core_info()
    nc, ns = info.num_cores, info.num_subcores
    nw = nc * ns
    units_per_w = N_UNITS // nw

    mesh = plsc.VectorSubcoreMesh(core_axis_name="c", subcore_axis_name="s")

    @functools.partial(
        pl.kernel,
        mesh=mesh,
        compiler_params=pltpu.CompilerParams(
            use_tc_tiling_on_sc=False, needs_layout_passes=False
        ),
        out_type=jax.ShapeDtypeStruct((T, EMB, B), jnp.float32),
        scratch_types=[
            pltpu.VMEM((CHUNK,), jnp.int32),
            pltpu.VMEM((CHUNK, EMB), jnp.float32),
            pltpu.VMEM((EMB, CHUNK), jnp.float32),
            pltpu.SemaphoreType.DMA,
        ],
    )
    def emb_kernel(table_hbm, xt_hbm, out_hbm, idx_v, rows_v, outb_v, sem):
        wid = lax.axis_index("s") * nc + lax.axis_index("c")
        lanes = lax.iota(jnp.int32, 16)

        def unit_body(i, _):
            u = wid + i * nw
            t = u // N_CHUNKS
            b0 = (u % N_CHUNKS) * CHUNK
            pltpu.sync_copy(xt_hbm.at[t, pl.ds(b0, CHUNK)], idx_v)
            copies = [
                pltpu.async_copy(
                    table_hbm.at[idx_v.at[pl.ds(j * RW, RW)]],
                    rows_v.at[pl.ds(j * RW, RW)],
                    sem,
                )
                for j in range(NSTREAM)
            ]
            for cp in copies:
                cp.wait()

            def tr_body(jj, _):
                row16 = jj * 16 + lanes
                for e in range(EMB):
                    col16 = jnp.full((16,), e, jnp.int32)
                    vals = plsc.load_gather(rows_v, [row16, col16])
                    outb_v[e, pl.ds(jj * 16, 16)] = vals
                return 0

            lax.fori_loop(0, CHUNK // 16, tr_body, 0)
            pltpu.sync_copy(outb_v, out_hbm.at[t, :, pl.ds(b0, CHUNK)])
            return 0

        lax.fori_loop(0, units_per_w, unit_body, 0)

    return emb_kernel


def kernel(x, table):
    xt = x.T
    outp = _build()(table, xt)
    return jnp.transpose(outp, (2, 0, 1))


# speedup vs baseline: 1.2263x; 1.2263x over previous
import functools

import jax
import jax.numpy as jnp
from jax import lax
from jax.experimental import pallas as pl
from jax.experimental.pallas import tpu as pltpu
from jax.experimental.pallas import tpu_sc as plsc

VOCAB = 1_000_000
EMB = 32
B = 16384
T = 50
CHUNK = 1024
N_CHUNKS = B // CHUNK
N_UNITS = T * N_CHUNKS
RW = 128
NSTREAM = CHUNK // RW


@functools.lru_cache(maxsize=1)
def _build():
    info = plsc.get_sparse_core_info()
    nc, ns = info.num_cores, info.num_subcores
    nw = nc * ns
    units_per_w = N_UNITS // nw

    mesh = plsc.VectorSubcoreMesh(core_axis_name="c", subcore_axis_name="s")

    @functools.partial(
        pl.kernel,
        mesh=mesh,
        compiler_params=pltpu.CompilerParams(
            use_tc_tiling_on_sc=False, needs_layout_passes=False
        ),
        out_type=jax.ShapeDtypeStruct((T, EMB, B), jnp.float32),
        scratch_types=[
            pltpu.VMEM((CHUNK,), jnp.int32),
            pltpu.VMEM((CHUNK, EMB), jnp.float32),
            pltpu.VMEM((EMB, CHUNK), jnp.float32),
            pltpu.SemaphoreType.DMA,
        ],
    )
    def emb_kernel(table_hbm, xt_hbm, out_hbm, idx_v, rows_v, outb_v, sem):
        wid = lax.axis_index("s") * nc + lax.axis_index("c")
        lanes = lax.iota(jnp.int32, 16)

        def unit_body(i, _):
            u = wid + i * nw
            t = u // N_CHUNKS
            b0 = (u % N_CHUNKS) * CHUNK
            pltpu.sync_copy(xt_hbm.at[t, pl.ds(b0, CHUNK)], idx_v)
            copies = [
                pltpu.async_copy(
                    table_hbm.at[idx_v.at[pl.ds(j * RW, RW)]],
                    rows_v.at[pl.ds(j * RW, RW)],
                    sem,
                )
                for j in range(NSTREAM)
            ]
            for cp in copies:
                cp.wait()

            @plsc.parallel_loop(0, CHUNK // 16, unroll=4)
            def tr_body(jj):
                row16 = jj * 16 + lanes
                for e0 in range(0, EMB, 4):
                    vals = [
                        plsc.load_gather(
                            rows_v, [row16, jnp.full((16,), e, jnp.int32)]
                        )
                        for e in range(e0, e0 + 4)
                    ]
                    for k, e in enumerate(range(e0, e0 + 4)):
                        outb_v[e, pl.ds(jj * 16, 16)] = vals[k]
            pltpu.sync_copy(outb_v, out_hbm.at[t, :, pl.ds(b0, CHUNK)])
            return 0

        lax.fori_loop(0, units_per_w, unit_body, 0)

    return emb_kernel


def kernel(x, table):
    xt = x.T
    outp = _build()(table, xt)
    return jnp.transpose(outp, (2, 0, 1))


# speedup vs baseline: 1.7373x; 1.4167x over previous
import functools

import jax
import jax.numpy as jnp
from jax import lax
from jax.experimental import pallas as pl
from jax.experimental.pallas import tpu as pltpu
from jax.experimental.pallas import tpu_sc as plsc

VOCAB = 1_000_000
EMB = 32
B = 16384
T = 50
CHUNK = 1024
N_CHUNKS = B // CHUNK
N_UNITS = T * N_CHUNKS
RW = 128
NSTREAM = CHUNK // RW


@functools.lru_cache(maxsize=1)
def _build():
    info = plsc.get_sparse_core_info()
    nc, ns = info.num_cores, info.num_subcores
    nw = nc * ns
    units_per_w = N_UNITS // nw

    mesh = plsc.VectorSubcoreMesh(core_axis_name="c", subcore_axis_name="s")

    @functools.partial(
        pl.kernel,
        mesh=mesh,
        compiler_params=pltpu.CompilerParams(
            use_tc_tiling_on_sc=False, needs_layout_passes=False
        ),
        out_type=jax.ShapeDtypeStruct((T, EMB, B), jnp.float32),
        scratch_types=[
            pltpu.VMEM((CHUNK,), jnp.int32),
            pltpu.VMEM((CHUNK, EMB), jnp.float32),
            pltpu.VMEM((EMB, CHUNK), jnp.float32),
            pltpu.SemaphoreType.DMA,
        ],
    )
    def emb_kernel(table_hbm, xt_hbm, out_hbm, idx_v, rows_v, outb_v, sem):
        wid = lax.axis_index("s") * nc + lax.axis_index("c")
        lanes = lax.iota(jnp.int32, 16)


        def unit_body(i, _):
            u = wid + i * nw
            t = u // N_CHUNKS
            b0 = (u % N_CHUNKS) * CHUNK
            pltpu.sync_copy(xt_hbm.at[t, pl.ds(b0, CHUNK)], idx_v)
            copies = [
                pltpu.async_copy(
                    table_hbm.at[idx_v.at[pl.ds(j * RW, RW)]],
                    rows_v.at[pl.ds(j * RW, RW)],
                    sem,
                )
                for j in range(NSTREAM)
            ]
            for cp in copies:
                cp.wait()

            @plsc.parallel_loop(0, CHUNK // 16, unroll=4)
            def tr_body(jj):
                row16 = jj * 16 + lanes
                ecol = lanes
                for _ in range(EMB):
                    vals = plsc.load_gather(rows_v, [row16, ecol])
                    plsc.store_scatter(outb_v, [ecol, row16], vals)
                    ecol = jnp.bitwise_and(ecol + 1, EMB - 1)
            pltpu.sync_copy(outb_v, out_hbm.at[t, :, pl.ds(b0, CHUNK)])
            return 0

        lax.fori_loop(0, units_per_w, unit_body, 0)

    return emb_kernel


def kernel(x, table):
    xt = x.T
    outp = _build()(table, xt)
    return jnp.transpose(outp, (2, 0, 1))
